# baseline (speedup 1.0000x reference)
import jax
import jax.numpy as jnp
from jax import lax
from jax.experimental import pallas as pl
from jax.experimental.pallas import tpu as pltpu

B, L, D, F, T = 8, 512, 256, 256, 1536
TC = 512
NC = T // TC
LN_EPS = 1e-5


def _body(mel_ref, bl_ref, x_ref, tgt_ref,
          w1a, w1b, w1c, bc1_r, g1_r, b1_r,
          w2a, w2b, w2c, bc2_r, g2_r, b2_r,
          wl_r, out_ref, dur_ref, starts_s, ends_s, tot_s):
    j = pl.program_id(1)
    xb = x_ref[0]
    xbf = xb.astype(jnp.bfloat16)

    @pl.when(j == 0)
    def _():
        zrow = jnp.zeros((1, D), jnp.bfloat16)

        def ln_relu(h, g, b):
            m = jnp.mean(h, axis=-1, keepdims=True)
            v = jnp.mean((h - m) * (h - m), axis=-1, keepdims=True)
            hn = (h - m) * lax.rsqrt(v + LN_EPS)
            return jnp.maximum(hn * g + b, 0.0)

        def conv(a_bf, wu, wc, wd, bias):
            up = jnp.concatenate([zrow, a_bf[:-1, :]], axis=0)
            dn = jnp.concatenate([a_bf[1:, :], zrow], axis=0)
            mm = lambda t, w: jnp.dot(t, w[...],
                                      preferred_element_type=jnp.float32)
            return mm(up, wu) + mm(a_bf, wc) + mm(dn, wd) + bias[...]

        h = ln_relu(conv(xbf, w1a, w1b, w1c, bc1_r), g1_r[...], b1_r[...])
        h2 = ln_relu(conv(h.astype(jnp.bfloat16), w2a, w2b, w2c, bc2_r),
                     g2_r[...], b2_r[...])
        dur = jnp.sum(h2 * wl_r[...], axis=-1, keepdims=True) + bl_ref[0]
        dur_ref[0] = jnp.maximum(dur, 0.0)

        ia = lax.broadcasted_iota(jnp.int32, (L, L), 0)
        ib = lax.broadcasted_iota(jnp.int32, (L, L), 1)
        m_tri = (ia <= ib).astype(jnp.float32)
        tgt_row = tgt_ref[0]
        ends_row = jax.lax.dot(tgt_row, m_tri,
                               precision=jax.lax.Precision.HIGHEST)
        mel_f = mel_ref[0].astype(jnp.float32)
        starts_s[...] = ends_row - tgt_row
        ends_s[...] = jnp.minimum(ends_row, mel_f)
        tot_s[0] = jnp.minimum(jnp.max(ends_row), mel_f)

    live = (j * TC).astype(jnp.float32) < tot_s[0]

    @pl.when(live)
    def _():
        t_f = (lax.broadcasted_iota(jnp.int32, (TC, 1), 0)
               + j * TC).astype(jnp.float32)
        a_mat = ((t_f >= starts_s[...]) & (t_f < ends_s[...])
                 ).astype(jnp.bfloat16)
        out_ref[0] = jnp.dot(a_mat, xbf, preferred_element_type=jnp.float32)

    @pl.when(jnp.logical_not(live))
    def _():
        out_ref[0] = jnp.zeros((TC, D), jnp.float32)


def kernel(x, target, mel_max_length, Wc1, bc1, g1, b1, Wc2, bc2, g2, b2, Wl, bl):
    x = x.astype(jnp.float32)
    tgt3 = target.astype(jnp.float32).reshape(B, 1, L)
    mel = jnp.asarray(mel_max_length, jnp.int32).reshape(1)
    blv = bl.astype(jnp.float32).reshape(1)

    row = lambda a: a.astype(jnp.float32).reshape(1, F)
    wmat = lambda W, k: jnp.transpose(W[:, :, k]).astype(jnp.bfloat16)

    full = lambda shp: pl.BlockSpec(shp, lambda i, j: (0,) * len(shp))
    out, dur3 = pl.pallas_call(
        _body,
        grid=(B, NC),
        in_specs=[
            pl.BlockSpec(memory_space=pltpu.SMEM),
            pl.BlockSpec(memory_space=pltpu.SMEM),
            pl.BlockSpec((1, L, D), lambda i, j: (i, 0, 0)),
            pl.BlockSpec((1, 1, L), lambda i, j: (i, 0, 0)),
            full((D, F)), full((D, F)), full((D, F)),
            full((1, F)), full((1, F)), full((1, F)),
            full((F, F)), full((F, F)), full((F, F)),
            full((1, F)), full((1, F)), full((1, F)),
            full((1, F)),
        ],
        out_specs=[
            pl.BlockSpec((1, TC, D), lambda i, j: (i, j, 0)),
            pl.BlockSpec((1, L, 1), lambda i, j: (i, 0, 0)),
        ],
        out_shape=[
            jax.ShapeDtypeStruct((B, T, D), jnp.float32),
            jax.ShapeDtypeStruct((B, L, 1), jnp.float32),
        ],
        scratch_shapes=[
            pltpu.VMEM((1, L), jnp.float32),
            pltpu.VMEM((1, L), jnp.float32),
            pltpu.SMEM((1,), jnp.float32),
        ],
    )(mel, blv, x, tgt3,
      wmat(Wc1, 0), wmat(Wc1, 1), wmat(Wc1, 2), row(bc1), row(g1), row(b1),
      wmat(Wc2, 0), wmat(Wc2, 1), wmat(Wc2, 2), row(bc2), row(g2), row(b2),
      Wl.astype(jnp.float32).reshape(1, F))

    return (out, dur3.reshape(B, L))


# speedup vs baseline: 1.0302x; 1.0302x over previous
import jax
import jax.numpy as jnp
from jax import lax
from jax.experimental import pallas as pl
from jax.experimental.pallas import tpu as pltpu

B, L, D, F, T = 8, 512, 256, 256, 1536
TC = 512
NC = T // TC
LN_EPS = 1e-5


def _pred_body(mel_ref, bl_ref, x_ref, tgt_ref,
               w1a, w1b, w1c, bc1_r, g1_r, b1_r,
               w2a, w2b, w2c, bc2_r, g2_r, b2_r,
               wl_r, dur_ref, starts_ref, ends_ref):
    xb = x_ref[0]
    xbf = xb.astype(jnp.bfloat16)
    zrow = jnp.zeros((1, D), jnp.bfloat16)

    def ln_relu(h, g, b):
        m = jnp.mean(h, axis=-1, keepdims=True)
        v = jnp.mean((h - m) * (h - m), axis=-1, keepdims=True)
        hn = (h - m) * lax.rsqrt(v + LN_EPS)
        return jnp.maximum(hn * g + b, 0.0)

    def conv(a_bf, wu, wc, wd, bias):
        up = jnp.concatenate([zrow, a_bf[:-1, :]], axis=0)
        dn = jnp.concatenate([a_bf[1:, :], zrow], axis=0)
        mm = lambda t, w: jnp.dot(t, w[...],
                                  preferred_element_type=jnp.float32)
        return mm(up, wu) + mm(a_bf, wc) + mm(dn, wd) + bias[...]

    h = ln_relu(conv(xbf, w1a, w1b, w1c, bc1_r), g1_r[...], b1_r[...])
    h2 = ln_relu(conv(h.astype(jnp.bfloat16), w2a, w2b, w2c, bc2_r),
                 g2_r[...], b2_r[...])
    dur = jnp.sum(h2 * wl_r[...], axis=-1, keepdims=True) + bl_ref[0]
    dur_ref[0] = jnp.maximum(dur, 0.0)

    ia = lax.broadcasted_iota(jnp.int32, (L, L), 0)
    ib = lax.broadcasted_iota(jnp.int32, (L, L), 1)
    m_tri = (ia <= ib).astype(jnp.float32)
    tgt_row = tgt_ref[0]
    ends_row = jax.lax.dot(tgt_row, m_tri, precision=jax.lax.Precision.HIGHEST)
    starts_ref[0] = ends_row - tgt_row
    ends_ref[0] = jnp.minimum(ends_row, mel_ref[0].astype(jnp.float32))


def _align_body(x_ref, starts_ref, ends_ref, out_ref):
    j = pl.program_id(1)
    t_f = (lax.broadcasted_iota(jnp.int32, (TC, 1), 0)
           + j * TC).astype(jnp.float32)
    a_mat = ((t_f >= starts_ref[0]) & (t_f < ends_ref[0])
             ).astype(jnp.bfloat16)
    out_ref[0] = jnp.dot(a_mat, x_ref[0].astype(jnp.bfloat16),
                         preferred_element_type=jnp.float32)


def kernel(x, target, mel_max_length, Wc1, bc1, g1, b1, Wc2, bc2, g2, b2, Wl, bl):
    x = x.astype(jnp.float32)
    tgt3 = target.astype(jnp.float32).reshape(B, 1, L)
    mel = jnp.asarray(mel_max_length, jnp.int32).reshape(1)
    blv = bl.astype(jnp.float32).reshape(1)

    row = lambda a: a.astype(jnp.float32).reshape(1, F)
    wmat = lambda W, k: jnp.transpose(W[:, :, k]).astype(jnp.bfloat16)

    full = lambda shp: pl.BlockSpec(shp, lambda i: (0,) * len(shp))
    dur3, starts3, ends3 = pl.pallas_call(
        _pred_body,
        grid=(B,),
        in_specs=[
            pl.BlockSpec(memory_space=pltpu.SMEM),
            pl.BlockSpec(memory_space=pltpu.SMEM),
            pl.BlockSpec((1, L, D), lambda i: (i, 0, 0)),
            pl.BlockSpec((1, 1, L), lambda i: (i, 0, 0)),
            full((D, F)), full((D, F)), full((D, F)),
            full((1, F)), full((1, F)), full((1, F)),
            full((F, F)), full((F, F)), full((F, F)),
            full((1, F)), full((1, F)), full((1, F)),
            full((1, F)),
        ],
        out_specs=[
            pl.BlockSpec((1, L, 1), lambda i: (i, 0, 0)),
            pl.BlockSpec((1, 1, L), lambda i: (i, 0, 0)),
            pl.BlockSpec((1, 1, L), lambda i: (i, 0, 0)),
        ],
        out_shape=[
            jax.ShapeDtypeStruct((B, L, 1), jnp.float32),
            jax.ShapeDtypeStruct((B, 1, L), jnp.float32),
            jax.ShapeDtypeStruct((B, 1, L), jnp.float32),
        ],
    )(mel, blv, x, tgt3,
      wmat(Wc1, 0), wmat(Wc1, 1), wmat(Wc1, 2), row(bc1), row(g1), row(b1),
      wmat(Wc2, 0), wmat(Wc2, 1), wmat(Wc2, 2), row(bc2), row(g2), row(b2),
      Wl.astype(jnp.float32).reshape(1, F))

    out = pl.pallas_call(
        _align_body,
        grid=(B, NC),
        in_specs=[
            pl.BlockSpec((1, L, D), lambda i, j: (i, 0, 0)),
            pl.BlockSpec((1, 1, L), lambda i, j: (i, 0, 0)),
            pl.BlockSpec((1, 1, L), lambda i, j: (i, 0, 0)),
        ],
        out_specs=pl.BlockSpec((1, TC, D), lambda i, j: (i, j, 0)),
        out_shape=jax.ShapeDtypeStruct((B, T, D), jnp.float32),
    )(x, starts3, ends3)

    return (out, dur3.reshape(B, L))


# speedup vs baseline: 1.4993x; 1.4553x over previous
import jax
import jax.numpy as jnp
from jax import lax
from jax.experimental import pallas as pl
from jax.experimental.pallas import tpu as pltpu

B, L, D, F, T = 8, 512, 256, 256, 1536
LN_EPS = 1e-5


def _body(mel_ref, bl_ref, x_ref, tgt_ref,
          w1a, w1b, w1c, bc1_r, g1_r, b1_r,
          w2a, w2b, w2c, bc2_r, g2_r, b2_r,
          wl_r, out_ref, dur_ref):
    xb = x_ref[0]
    xbf = xb.astype(jnp.bfloat16)
    zrow = jnp.zeros((1, D), jnp.bfloat16)

    def ln_relu(h, g, b):
        m = jnp.mean(h, axis=-1, keepdims=True)
        v = jnp.mean((h - m) * (h - m), axis=-1, keepdims=True)
        hn = (h - m) * lax.rsqrt(v + LN_EPS)
        return jnp.maximum(hn * g + b, 0.0)

    def conv(a_bf, wu, wc, wd, bias):
        up = jnp.concatenate([zrow, a_bf[:-1, :]], axis=0)
        dn = jnp.concatenate([a_bf[1:, :], zrow], axis=0)
        mm = lambda t, w: jnp.dot(t, w[...],
                                  preferred_element_type=jnp.float32)
        return mm(up, wu) + mm(a_bf, wc) + mm(dn, wd) + bias[...]

    h = ln_relu(conv(xbf, w1a, w1b, w1c, bc1_r), g1_r[...], b1_r[...])
    h2 = ln_relu(conv(h.astype(jnp.bfloat16), w2a, w2b, w2c, bc2_r),
                 g2_r[...], b2_r[...])
    dur = jnp.sum(h2 * wl_r[...], axis=-1, keepdims=True) + bl_ref[0]
    dur_ref[0] = jnp.maximum(dur, 0.0)

    ia = lax.broadcasted_iota(jnp.int32, (L, L), 0)
    ib = lax.broadcasted_iota(jnp.int32, (L, L), 1)
    m_tri = (ia <= ib).astype(jnp.float32)
    tgt_row = tgt_ref[0]
    ends_row = jax.lax.dot(tgt_row, m_tri, precision=jax.lax.Precision.HIGHEST)
    starts_row = ends_row - tgt_row
    ends_row = jnp.minimum(ends_row, mel_ref[0].astype(jnp.float32))

    t_f = lax.broadcasted_iota(jnp.int32, (T, 1), 0).astype(jnp.float32)
    a_mat = ((t_f >= starts_row) & (t_f < ends_row)).astype(jnp.bfloat16)
    out_ref[0] = jnp.dot(a_mat, xbf, preferred_element_type=jnp.float32)


def kernel(x, target, mel_max_length, Wc1, bc1, g1, b1, Wc2, bc2, g2, b2, Wl, bl):
    x = x.astype(jnp.float32)
    tgt3 = target.astype(jnp.float32).reshape(B, 1, L)
    mel = jnp.asarray(mel_max_length, jnp.int32).reshape(1)
    blv = bl.astype(jnp.float32).reshape(1)

    row = lambda a: a.astype(jnp.float32).reshape(1, F)
    wmat = lambda W, k: jnp.transpose(W[:, :, k]).astype(jnp.bfloat16)

    full = lambda shp: pl.BlockSpec(shp, lambda i: (0,) * len(shp))
    out, dur3 = pl.pallas_call(
        _body,
        grid=(B,),
        in_specs=[
            pl.BlockSpec(memory_space=pltpu.SMEM),
            pl.BlockSpec(memory_space=pltpu.SMEM),
            pl.BlockSpec((1, L, D), lambda i: (i, 0, 0)),
            pl.BlockSpec((1, 1, L), lambda i: (i, 0, 0)),
            full((D, F)), full((D, F)), full((D, F)),
            full((1, F)), full((1, F)), full((1, F)),
            full((F, F)), full((F, F)), full((F, F)),
            full((1, F)), full((1, F)), full((1, F)),
            full((1, F)),
        ],
        out_specs=[
            pl.BlockSpec((1, T, D), lambda i: (i, 0, 0)),
            pl.BlockSpec((1, L, 1), lambda i: (i, 0, 0)),
        ],
        out_shape=[
            jax.ShapeDtypeStruct((B, T, D), jnp.float32),
            jax.ShapeDtypeStruct((B, L, 1), jnp.float32),
        ],
        compiler_params=pltpu.CompilerParams(
            dimension_semantics=("parallel",)),
    )(mel, blv, x, tgt3,
      wmat(Wc1, 0), wmat(Wc1, 1), wmat(Wc1, 2), row(bc1), row(g1), row(b1),
      wmat(Wc2, 0), wmat(Wc2, 1), wmat(Wc2, 2), row(bc2), row(g2), row(b2),
      Wl.astype(jnp.float32).reshape(1, F))

    return (out, dur3.reshape(B, L))


# speedup vs baseline: 1.5037x; 1.0029x over previous
import jax
import jax.numpy as jnp
from jax import lax
from jax.experimental import pallas as pl
from jax.experimental.pallas import tpu as pltpu

B, L, D, F, T = 8, 512, 256, 256, 1536
LN_EPS = 1e-5


def _body(mel_ref, bl_ref, x_ref, tgt_ref,
          w1a, w1b, w1c, bc1_r, g1_r, b1_r,
          w2a, w2b, w2c, bc2_r, g2_r, b2_r,
          wl_r, out_ref, dur_ref):
    xb = x_ref[0]
    xbf = xb.astype(jnp.bfloat16)
    zrow = jnp.zeros((1, D), jnp.bfloat16)

    def ln_relu(h, g, b):
        m = jnp.mean(h, axis=-1, keepdims=True)
        v = jnp.mean((h - m) * (h - m), axis=-1, keepdims=True)
        hn = (h - m) * lax.rsqrt(v + LN_EPS)
        return jnp.maximum(hn * g + b, 0.0)

    def conv(a_bf, wu, wc, wd, bias):
        up = jnp.concatenate([zrow, a_bf[:-1, :]], axis=0)
        dn = jnp.concatenate([a_bf[1:, :], zrow], axis=0)
        mm = lambda t, w: jnp.dot(t, w[...],
                                  preferred_element_type=jnp.float32)
        return mm(up, wu) + mm(a_bf, wc) + mm(dn, wd) + bias[...]

    h = ln_relu(conv(xbf, w1a, w1b, w1c, bc1_r), g1_r[...], b1_r[...])
    h2 = ln_relu(conv(h.astype(jnp.bfloat16), w2a, w2b, w2c, bc2_r),
                 g2_r[...], b2_r[...])
    dur = jnp.sum(h2 * wl_r[...], axis=-1, keepdims=True) + bl_ref[0]
    dur_ref[0] = jnp.maximum(dur, 0.0)

    ia = lax.broadcasted_iota(jnp.int32, (L, L), 0)
    ib = lax.broadcasted_iota(jnp.int32, (L, L), 1)
    m_tri = (ia <= ib).astype(jnp.float32)
    tgt_row = tgt_ref[0]
    ends_row = jax.lax.dot(tgt_row, m_tri, precision=jax.lax.Precision.HIGHEST)
    starts_row = ends_row - tgt_row
    ends_row = jnp.minimum(ends_row, mel_ref[0].astype(jnp.float32))

    t_f = lax.broadcasted_iota(jnp.int32, (T, 1), 0).astype(jnp.float32)
    a_mat = ((t_f >= starts_row) & (t_f < ends_row)).astype(jnp.bfloat16)
    out_ref[0] = jnp.dot(a_mat, xbf, preferred_element_type=jnp.float32)


def kernel(x, target, mel_max_length, Wc1, bc1, g1, b1, Wc2, bc2, g2, b2, Wl, bl):
    x = x.astype(jnp.float32)
    tgt3 = target.astype(jnp.float32).reshape(B, 1, L)
    mel = jnp.asarray(mel_max_length, jnp.int32).reshape(1)
    blv = bl.astype(jnp.float32).reshape(1)

    row = lambda a: a.astype(jnp.float32).reshape(1, F)
    wmat = lambda W, k: jnp.transpose(W[:, :, k]).astype(jnp.bfloat16)

    full = lambda shp: pl.BlockSpec(shp, lambda i: (0,) * len(shp))
    out, dur3 = pl.pallas_call(
        _body,
        grid=(B,),
        in_specs=[
            pl.BlockSpec(memory_space=pltpu.SMEM),
            pl.BlockSpec(memory_space=pltpu.SMEM),
            pl.BlockSpec((1, L, D), lambda i: (i, 0, 0)),
            pl.BlockSpec((1, 1, L), lambda i: (i, 0, 0)),
            full((D, F)), full((D, F)), full((D, F)),
            full((1, F)), full((1, F)), full((1, F)),
            full((F, F)), full((F, F)), full((F, F)),
            full((1, F)), full((1, F)), full((1, F)),
            full((1, F)),
        ],
        out_specs=[
            pl.BlockSpec((1, T, D), lambda i: (i, 0, 0)),
            pl.BlockSpec((1, L, 1), lambda i: (i, 0, 0)),
        ],
        out_shape=[
            jax.ShapeDtypeStruct((B, T, D), jnp.float32),
            jax.ShapeDtypeStruct((B, L, 1), jnp.float32),
        ],
    )(mel, blv, x, tgt3,
      wmat(Wc1, 0), wmat(Wc1, 1), wmat(Wc1, 2), row(bc1), row(g1), row(b1),
      wmat(Wc2, 0), wmat(Wc2, 1), wmat(Wc2, 2), row(bc2), row(g2), row(b2),
      Wl.astype(jnp.float32).reshape(1, F))

    return (out, dur3.reshape(B, L))


# speedup vs baseline: 1.5330x; 1.0195x over previous
import jax
import jax.numpy as jnp
from jax import lax
from jax.experimental import pallas as pl
from jax.experimental.pallas import tpu as pltpu

B, L, D, F, T = 8, 512, 256, 256, 1536
LN_EPS = 1e-5


def _body(mel_ref, bl_ref, x_ref, tgt_ref,
          w1a, w1b, w1c, bc1_r, g1_r, b1_r,
          w2a, w2b, w2c, bc2_r, g2_r, b2_r,
          wl_r, out_ref, dur_ref):
    xb = x_ref[0]
    xbf = xb.astype(jnp.bfloat16)
    zrow = jnp.zeros((1, D), jnp.bfloat16)

    def ln_relu(h, g, b):
        m = jnp.mean(h, axis=-1, keepdims=True)
        v = jnp.mean((h - m) * (h - m), axis=-1, keepdims=True)
        hn = (h - m) * lax.rsqrt(v + LN_EPS)
        return jnp.maximum(hn * g + b, 0.0)

    def conv(a_bf, wu, wc, wd, bias):
        up = jnp.concatenate([zrow, a_bf[:-1, :]], axis=0)
        dn = jnp.concatenate([a_bf[1:, :], zrow], axis=0)
        mm = lambda t, w: jnp.dot(t, w[...],
                                  preferred_element_type=jnp.float32)
        return mm(up, wu) + mm(a_bf, wc) + mm(dn, wd) + bias[...]

    h = ln_relu(conv(xbf, w1a, w1b, w1c, bc1_r), g1_r[...], b1_r[...])
    h2 = ln_relu(conv(h.astype(jnp.bfloat16), w2a, w2b, w2c, bc2_r),
                 g2_r[...], b2_r[...])
    dur = jnp.sum(h2 * wl_r[...], axis=-1, keepdims=True) + bl_ref[0]
    dur_ref[0] = jnp.maximum(dur, 0.0)

    tgt_row = tgt_ref[0]
    lane = lax.broadcasted_iota(jnp.int32, (1, L), 1)
    ends_row = tgt_row
    k = 1
    while k < L:
        rolled = pltpu.roll(ends_row, k, axis=1)
        ends_row = ends_row + jnp.where(lane >= k, rolled, 0.0)
        k *= 2
    starts_row = ends_row - tgt_row
    ends_row = jnp.minimum(ends_row, mel_ref[0].astype(jnp.float32))

    t_f = lax.broadcasted_iota(jnp.int32, (T, 1), 0).astype(jnp.float32)
    a_mat = ((t_f >= starts_row) & (t_f < ends_row)).astype(jnp.bfloat16)
    out_ref[0] = jnp.dot(a_mat, xbf, preferred_element_type=jnp.float32)


def kernel(x, target, mel_max_length, Wc1, bc1, g1, b1, Wc2, bc2, g2, b2, Wl, bl):
    x = x.astype(jnp.float32)
    tgt3 = target.astype(jnp.float32).reshape(B, 1, L)
    mel = jnp.asarray(mel_max_length, jnp.int32).reshape(1)
    blv = bl.astype(jnp.float32).reshape(1)

    row = lambda a: a.astype(jnp.float32).reshape(1, F)
    wmat = lambda W, k: jnp.transpose(W[:, :, k]).astype(jnp.bfloat16)

    full = lambda shp: pl.BlockSpec(shp, lambda i: (0,) * len(shp))
    out, dur3 = pl.pallas_call(
        _body,
        grid=(B,),
        in_specs=[
            pl.BlockSpec(memory_space=pltpu.SMEM),
            pl.BlockSpec(memory_space=pltpu.SMEM),
            pl.BlockSpec((1, L, D), lambda i: (i, 0, 0)),
            pl.BlockSpec((1, 1, L), lambda i: (i, 0, 0)),
            full((D, F)), full((D, F)), full((D, F)),
            full((1, F)), full((1, F)), full((1, F)),
            full((F, F)), full((F, F)), full((F, F)),
            full((1, F)), full((1, F)), full((1, F)),
            full((1, F)),
        ],
        out_specs=[
            pl.BlockSpec((1, T, D), lambda i: (i, 0, 0)),
            pl.BlockSpec((1, L, 1), lambda i: (i, 0, 0)),
        ],
        out_shape=[
            jax.ShapeDtypeStruct((B, T, D), jnp.float32),
            jax.ShapeDtypeStruct((B, L, 1), jnp.float32),
        ],
    )(mel, blv, x, tgt3,
      wmat(Wc1, 0), wmat(Wc1, 1), wmat(Wc1, 2), row(bc1), row(g1), row(b1),
      wmat(Wc2, 0), wmat(Wc2, 1), wmat(Wc2, 2), row(bc2), row(g2), row(b2),
      Wl.astype(jnp.float32).reshape(1, F))

    return (out, dur3.reshape(B, L))


# speedup vs baseline: 1.5831x; 1.0327x over previous
import jax
import jax.numpy as jnp
from jax import lax
from jax.experimental import pallas as pl
from jax.experimental.pallas import tpu as pltpu

B, L, D, F, T = 8, 512, 256, 256, 1536
LN_EPS = 1e-5


def _body(mel_ref, bl_ref, x_ref, tgt_ref,
          w1a, w1b, w1c, bc1_r, g1_r, b1_r,
          w2a, w2b, w2c, bc2_r, g2_r, b2_r,
          wl_r, out_ref, dur_ref):
    xb = x_ref[0]
    xbf = xb.astype(jnp.bfloat16)
    zrow = jnp.zeros((1, D), jnp.bfloat16)

    def ln_relu(h, g, b):
        m = jnp.mean(h, axis=-1, keepdims=True)
        s2 = jnp.mean(h * h, axis=-1, keepdims=True)
        v = s2 - m * m
        hn = (h - m) * lax.rsqrt(v + LN_EPS)
        return jnp.maximum(hn * g + b, 0.0)

    def conv(a_bf, wu, wc, wd, bias):
        up = jnp.concatenate([zrow, a_bf[:-1, :]], axis=0)
        dn = jnp.concatenate([a_bf[1:, :], zrow], axis=0)
        mm = lambda t, w: jnp.dot(t, w[...],
                                  preferred_element_type=jnp.float32)
        return mm(up, wu) + mm(a_bf, wc) + mm(dn, wd) + bias[...]

    h = ln_relu(conv(xbf, w1a, w1b, w1c, bc1_r), g1_r[...], b1_r[...])
    h2 = ln_relu(conv(h.astype(jnp.bfloat16), w2a, w2b, w2c, bc2_r),
                 g2_r[...], b2_r[...])
    dur = jnp.sum(h2 * wl_r[...], axis=-1, keepdims=True) + bl_ref[0]
    dur_ref[0] = jnp.maximum(dur, 0.0)

    tgt_row = tgt_ref[0]
    lane = lax.broadcasted_iota(jnp.int32, (1, L), 1)
    ends_row = tgt_row
    k = 1
    while k < L:
        rolled = pltpu.roll(ends_row, k, axis=1)
        ends_row = ends_row + jnp.where(lane >= k, rolled, 0.0)
        k *= 2
    starts_row = ends_row - tgt_row
    ends_row = jnp.minimum(ends_row, mel_ref[0].astype(jnp.float32))

    t_f = lax.broadcasted_iota(jnp.int32, (T, 1), 0).astype(jnp.float32)
    a_mat = ((t_f >= starts_row) & (t_f < ends_row)).astype(jnp.bfloat16)
    out_ref[0] = jnp.dot(a_mat, xbf, preferred_element_type=jnp.float32)


def kernel(x, target, mel_max_length, Wc1, bc1, g1, b1, Wc2, bc2, g2, b2, Wl, bl):
    x = x.astype(jnp.float32)
    tgt3 = target.astype(jnp.float32).reshape(B, 1, L)
    mel = jnp.asarray(mel_max_length, jnp.int32).reshape(1)
    blv = bl.astype(jnp.float32).reshape(1)

    row = lambda a: a.astype(jnp.float32).reshape(1, F)
    wmat = lambda W, k: jnp.transpose(W[:, :, k]).astype(jnp.bfloat16)

    full = lambda shp: pl.BlockSpec(shp, lambda i: (0,) * len(shp))
    out, dur3 = pl.pallas_call(
        _body,
        grid=(B,),
        in_specs=[
            pl.BlockSpec(memory_space=pltpu.SMEM),
            pl.BlockSpec(memory_space=pltpu.SMEM),
            pl.BlockSpec((1, L, D), lambda i: (i, 0, 0)),
            pl.BlockSpec((1, 1, L), lambda i: (i, 0, 0)),
            full((D, F)), full((D, F)), full((D, F)),
            full((1, F)), full((1, F)), full((1, F)),
            full((F, F)), full((F, F)), full((F, F)),
            full((1, F)), full((1, F)), full((1, F)),
            full((1, F)),
        ],
        out_specs=[
            pl.BlockSpec((1, T, D), lambda i: (i, 0, 0)),
            pl.BlockSpec((1, L, 1), lambda i: (i, 0, 0)),
        ],
        out_shape=[
            jax.ShapeDtypeStruct((B, T, D), jnp.float32),
            jax.ShapeDtypeStruct((B, L, 1), jnp.float32),
        ],
    )(mel, blv, x, tgt3,
      wmat(Wc1, 0), wmat(Wc1, 1), wmat(Wc1, 2), row(bc1), row(g1), row(b1),
      wmat(Wc2, 0), wmat(Wc2, 1), wmat(Wc2, 2), row(bc2), row(g2), row(b2),
      Wl.astype(jnp.float32).reshape(1, F))

    return (out, dur3.reshape(B, L))
